# baseline (speedup 1.0000x reference)
import functools

import jax
import jax.numpy as jnp
from jax.experimental import pallas as pl

_D = 128
_BLK = 512


def _proj_kernel(h_ref, w1_ref, a1_ref, w2_ref, a2_ref,
                 z1_ref, s1_ref, z2_ref, s2_ref):
    h = h_ref[...]
    z1 = jnp.dot(h, w1_ref[...], preferred_element_type=jnp.float32)
    z2 = jnp.dot(h, w2_ref[...], preferred_element_type=jnp.float32)
    z1_ref[...] = z1
    z2_ref[...] = z2
    s1_ref[...] = jnp.dot(z1, a1_ref[...], preferred_element_type=jnp.float32)
    s2_ref[...] = jnp.dot(z2, a2_ref[...], preferred_element_type=jnp.float32)


def _project(h, W1, a1, W2, a2):
    n = h.shape[0]
    a1m = jnp.concatenate([a1[:_D], a1[_D:]], axis=1)
    a2m = jnp.concatenate([a2[:_D], a2[_D:]], axis=1)
    return pl.pallas_call(
        _proj_kernel,
        grid=(pl.cdiv(n, _BLK),),
        in_specs=[
            pl.BlockSpec((_BLK, _D), lambda i: (i, 0)),
            pl.BlockSpec((_D, _D), lambda i: (0, 0)),
            pl.BlockSpec((_D, 2), lambda i: (0, 0)),
            pl.BlockSpec((_D, _D), lambda i: (0, 0)),
            pl.BlockSpec((_D, 2), lambda i: (0, 0)),
        ],
        out_specs=[
            pl.BlockSpec((_BLK, _D), lambda i: (i, 0)),
            pl.BlockSpec((_BLK, 2), lambda i: (i, 0)),
            pl.BlockSpec((_BLK, _D), lambda i: (i, 0)),
            pl.BlockSpec((_BLK, 2), lambda i: (i, 0)),
        ],
        out_shape=[
            jax.ShapeDtypeStruct((n, _D), jnp.float32),
            jax.ShapeDtypeStruct((n, 2), jnp.float32),
            jax.ShapeDtypeStruct((n, _D), jnp.float32),
            jax.ShapeDtypeStruct((n, 2), jnp.float32),
        ],
    )(h, W1, a1m, W2, a2m)


def _aggregate(z, s_src, s_dst, src, dst, num_nodes):
    e = jax.nn.leaky_relu(s_src[src] + s_dst[dst], negative_slope=0.01)
    e_max = jax.ops.segment_max(e, dst, num_segments=num_nodes)
    e_max = jnp.where(jnp.isfinite(e_max), e_max, 0.0)
    e_exp = jnp.exp(e - e_max[dst])
    denom = jax.ops.segment_sum(e_exp, dst, num_segments=num_nodes)
    alpha = e_exp / (denom[dst] + 1e-12)
    return jax.ops.segment_sum(alpha[:, None] * z[src], dst,
                               num_segments=num_nodes)


def _fusion_kernel(a_ref, b_ref, c_ref, w_ref, bias_ref, out_ref):
    A = a_ref[...]
    B = b_ref[...]
    C = c_ref[...]
    w = w_ref[...]
    bias = bias_ref[...]
    ta = jnp.dot(A, w, preferred_element_type=jnp.float32)
    tb = jnp.dot(B, w, preferred_element_type=jnp.float32)
    tc = jnp.dot(C, w, preferred_element_type=jnp.float32)
    score1 = ta[:, 0] + tb[:, 1] + bias[0, 0]
    score2 = ta[:, 2] + tc[:, 3] + bias[0, 1]
    m = jnp.maximum(score1, score2)
    p1 = jnp.exp(score1 - m)
    p2 = jnp.exp(score2 - m)
    denom = p1 + p2
    out_ref[...] = (A + (p1 / denom)[:, None] * B
                    + (p2 / denom)[:, None] * C)


def _fuse(A, B, C, w1, b1, w2, b2):
    n = A.shape[0]
    w = jnp.concatenate([w1[:_D], w1[_D:], w2[:_D], w2[_D:]], axis=1)
    bias = jnp.stack([b1, b2], axis=1)
    return pl.pallas_call(
        _fusion_kernel,
        grid=(pl.cdiv(n, _BLK),),
        in_specs=[
            pl.BlockSpec((_BLK, _D), lambda i: (i, 0)),
            pl.BlockSpec((_BLK, _D), lambda i: (i, 0)),
            pl.BlockSpec((_BLK, _D), lambda i: (i, 0)),
            pl.BlockSpec((_D, 4), lambda i: (0, 0)),
            pl.BlockSpec((1, 2), lambda i: (0, 0)),
        ],
        out_specs=pl.BlockSpec((_BLK, _D), lambda i: (i, 0)),
        out_shape=jax.ShapeDtypeStruct((n, _D), jnp.float32),
    )(A, B, C, w, bias)


@functools.partial(jax.jit)
def _kernel_impl(kn_emb, exer_emb, all_stu_emb, ek_edge_index, ue_edge_index,
                 W_ke, a_ke, W_ek, a_ek, W_ue, a_ue, W_eu, a_eu,
                 w1, b1, w2, b2):
    exer_n = exer_emb.shape[0]
    n_ek = exer_n + kn_emb.shape[0]
    n_ue = exer_n + all_stu_emb.shape[0]
    ek_src, ek_dst = ek_edge_index[0], ek_edge_index[1]
    ue_src, ue_dst = ue_edge_index[0], ue_edge_index[1]

    e_k_graph = jnp.concatenate([exer_emb, kn_emb], axis=0)
    z_ke, s_ke, z_ek, s_ek = _project(e_k_graph, W_ke, a_ke, W_ek, a_ek)
    k_from_e = _aggregate(z_ke, s_ke[:, 0], s_ke[:, 1], ek_src, ek_dst, n_ek)
    e_from_k = _aggregate(z_ek, s_ek[:, 0], s_ek[:, 1], ek_dst, ek_src, n_ek)

    e_u_graph = jnp.concatenate([exer_emb, all_stu_emb], axis=0)
    z_ue, s_ue, z_eu, s_eu = _project(e_u_graph, W_ue, a_ue, W_eu, a_eu)
    u_from_e = _aggregate(z_ue, s_ue[:, 0], s_ue[:, 1], ue_src, ue_dst, n_ue)
    e_from_u = _aggregate(z_eu, s_eu[:, 0], s_eu[:, 1], ue_dst, ue_src, n_ue)

    kn_out = kn_emb + k_from_e[exer_n:]
    stu_out = all_stu_emb + u_from_e[exer_n:]
    exer_out = _fuse(exer_emb, e_from_k[:exer_n], e_from_u[:exer_n],
                     w1, b1, w2, b2)
    return (kn_out, exer_out, stu_out)


def kernel(kn_emb, exer_emb, all_stu_emb, ek_edge_index, ue_edge_index,
           W_ke, a_ke, W_ek, a_ek, W_ue, a_ue, W_eu, a_eu, w1, b1, w2, b2):
    return _kernel_impl(kn_emb, exer_emb, all_stu_emb, ek_edge_index,
                        ue_edge_index, W_ke, a_ke, W_ek, a_ek, W_ue, a_ue,
                        W_eu, a_eu, w1, b1, w2, b2)
